# baseline (speedup 1.0000x reference)
import functools

import jax
import jax.numpy as jnp
from jax import lax
from jax.experimental import pallas as pl
from jax.experimental.pallas import tpu as pltpu
from jax.experimental.pallas import tpu_sc as plsc

B_ROWS = 4096
SEQ = 200
D = 64
PAIR = 2 * D
VOCAB = 1000000
B_TOTAL = B_ROWS * SEQ

NUM_CORES = 2
NUM_SUBCORES = 16
NW = NUM_CORES * NUM_SUBCORES
PER_W = B_TOTAL // NW
CHUNK = 128
N_CHUNKS = PER_W // CHUNK
NBUF = 4
OUTER = N_CHUNKS // NBUF

_mesh = plsc.VectorSubcoreMesh(core_axis_name="c", subcore_axis_name="s")


_PREP_BK = 16384
_PREP_GRID = -(-VOCAB // _PREP_BK)


def _prep_body(tnat_ref, out_ref):
    block = tnat_ref[...]
    out_ref[:, :D] = jnp.transpose(block, (1, 0))
    out_ref[:, D:] = jnp.zeros((_PREP_BK, D), jnp.float32)


_prep_table = pl.pallas_call(
    _prep_body,
    grid=(_PREP_GRID,),
    in_specs=[pl.BlockSpec((D, _PREP_BK), lambda i: (0, i))],
    out_specs=pl.BlockSpec((_PREP_BK, PAIR), lambda i: (i, 0)),
    out_shape=jax.ShapeDtypeStruct((VOCAB, PAIR), jnp.float32),
)


@functools.partial(
    pl.kernel,
    mesh=_mesh,
    out_type=jax.ShapeDtypeStruct((B_TOTAL, D), jnp.float32),
    scratch_types=(
        [pltpu.VMEM((PER_W,), jnp.int32)]
        + [pltpu.VMEM((CHUNK, D), jnp.float32) for _ in range(NBUF)]
        + [pltpu.SemaphoreType.DMA for _ in range(2 * NBUF)]
    ),
    compiler_params=pltpu.CompilerParams(use_tc_tiling_on_sc=False),
)
def _gather_rows(ids_hbm, table_hbm, out_hbm, idx_v, *bufs_and_sems):
    rows = bufs_and_sems[:NBUF]
    sg = bufs_and_sems[NBUF : 2 * NBUF]
    sw = bufs_and_sems[2 * NBUF : 3 * NBUF]

    wid = lax.axis_index("s") * NUM_CORES + lax.axis_index("c")
    base = wid * PER_W

    pltpu.sync_copy(ids_hbm.at[pl.ds(base, PER_W)], idx_v)

    def start_gather(j, b):
        pltpu.async_copy(
            table_hbm.at[idx_v.at[pl.ds(j * CHUNK, CHUNK)]], rows[b], sg[b]
        )

    def wait_gather(j, b):
        pltpu.make_async_copy(
            table_hbm.at[idx_v.at[pl.ds(j * CHUNK, CHUNK)]], rows[b], sg[b]
        ).wait()

    def start_write(j, b):
        pltpu.async_copy(rows[b], out_hbm.at[pl.ds(base + j * CHUNK, CHUNK)], sw[b])

    def wait_write(j, b):
        pltpu.make_async_copy(
            rows[b], out_hbm.at[pl.ds(base + j * CHUNK, CHUNK)], sw[b]
        ).wait()

    for b in range(NBUF):
        start_gather(b, b)

    def outer(o, carry):
        for b in range(NBUF):
            j = o * NBUF + b
            wait_gather(j, b)
            start_write(j, b)
            wait_write(j, b)
            start_gather(j + NBUF, b)
        return carry

    lax.fori_loop(0, OUTER - 1, outer, 0)

    for b in range(NBUF):
        j = (OUTER - 1) * NBUF + b
        wait_gather(j, b)
        start_write(j, b)
    for b in range(NBUF):
        j = (OUTER - 1) * NBUF + b
        wait_write(j, b)


def _select_transpose_body(pairs_ref, out_ref):
    pairs = pairs_ref[0]
    pt = jnp.transpose(pairs, (1, 0))
    lo = pt[:D, :]
    hi = pt[D:, :]
    out_ref[0] = jnp.stack([lo, hi], axis=2).reshape(D, B_ROWS // 4)


_select_transpose = pl.pallas_call(
    _select_transpose_body,
    grid=(SEQ, 4),
    in_specs=[
        pl.BlockSpec((1, B_ROWS // 8, PAIR), lambda t, c: (t, c, 0)),
    ],
    out_specs=pl.BlockSpec((1, D, B_ROWS // 4), lambda t, c: (t, 0, c)),
    out_shape=jax.ShapeDtypeStruct((SEQ, D, B_ROWS), jnp.float32),
)


def kernel(ids, emb_weight):
    ids_t = jnp.transpose(ids).astype(jnp.int32)
    tnat = jnp.transpose(emb_weight)
    table128 = _prep_table(tnat)
    table2 = table128.reshape(2 * VOCAB, D)
    rows = _gather_rows((ids_t * 2).reshape(-1), table2)
    planes = rows.reshape(SEQ, B_ROWS // 2, PAIR)
    out_t = _select_transpose(planes)
    return out_t.transpose(2, 0, 1)


# speedup vs baseline: 16.5034x; 16.5034x over previous
import functools

import jax
import jax.numpy as jnp
from jax import lax
from jax.experimental import pallas as pl
from jax.experimental.pallas import tpu as pltpu
from jax.experimental.pallas import tpu_sc as plsc

B_ROWS = 4096
SEQ = 200
D = 64
PAIR = 2 * D
VOCAB = 1000000
B_TOTAL = B_ROWS * SEQ

NUM_CORES = 2
NUM_SUBCORES = 16
NW = NUM_CORES * NUM_SUBCORES
PER_W = B_TOTAL // NW
CHUNK = 256
N_CHUNKS = PER_W // CHUNK
NBUF = 4
OUTER = N_CHUNKS // NBUF

_mesh = plsc.VectorSubcoreMesh(core_axis_name="c", subcore_axis_name="s")


_PREP_BK = 16384
_PREP_GRID = -(-VOCAB // _PREP_BK)


def _prep_body(tnat_ref, out_ref):
    block = tnat_ref[...]
    out_ref[:, :D] = jnp.transpose(block, (1, 0))
    out_ref[:, D:] = jnp.zeros((_PREP_BK, D), jnp.float32)


_prep_table = pl.pallas_call(
    _prep_body,
    grid=(_PREP_GRID,),
    in_specs=[pl.BlockSpec((D, _PREP_BK), lambda i: (0, i))],
    out_specs=pl.BlockSpec((_PREP_BK, PAIR), lambda i: (i, 0)),
    out_shape=jax.ShapeDtypeStruct((VOCAB, PAIR), jnp.float32),
)


@functools.partial(
    pl.kernel,
    mesh=_mesh,
    out_type=jax.ShapeDtypeStruct((B_TOTAL, 2, D), jnp.float32),
    scratch_types=(
        [pltpu.VMEM((PER_W,), jnp.int32)]
        + [pltpu.VMEM((CHUNK, D), jnp.float32) for _ in range(NBUF)]
        + [pltpu.SemaphoreType.DMA for _ in range(2 * NBUF)]
    ),
    compiler_params=pltpu.CompilerParams(use_tc_tiling_on_sc=False),
)
def _gather_rows(ids_hbm, table_hbm, out_hbm, idx_v, *bufs_and_sems):
    rows = bufs_and_sems[:NBUF]
    sg = bufs_and_sems[NBUF : 2 * NBUF]
    sw = bufs_and_sems[2 * NBUF : 3 * NBUF]

    wid = lax.axis_index("s") * NUM_CORES + lax.axis_index("c")
    base = wid * PER_W

    pltpu.sync_copy(ids_hbm.at[pl.ds(base, PER_W)], idx_v)

    def start_gather(j, b):
        pltpu.async_copy(
            table_hbm.at[idx_v.at[pl.ds(j * CHUNK, CHUNK)]], rows[b], sg[b]
        )

    def wait_gather(j, b):
        pltpu.make_async_copy(
            table_hbm.at[idx_v.at[pl.ds(j * CHUNK, CHUNK)]], rows[b], sg[b]
        ).wait()

    def start_write(j, b):
        pltpu.async_copy(
            rows[b], out_hbm.at[pl.ds(base + j * CHUNK, CHUNK), 0], sw[b]
        )

    def wait_write(j, b):
        pltpu.make_async_copy(
            rows[b], out_hbm.at[pl.ds(base + j * CHUNK, CHUNK), 0], sw[b]
        ).wait()

    for b in range(NBUF):
        start_gather(b, b)

    def outer(o, carry):
        for b in range(NBUF):
            j = o * NBUF + b
            wait_gather(j, b)
            start_write(j, b)
            wait_write(j, b)
            start_gather(j + NBUF, b)
        return carry

    lax.fori_loop(0, OUTER - 1, outer, 0)

    for b in range(NBUF):
        j = (OUTER - 1) * NBUF + b
        wait_gather(j, b)
        start_write(j, b)
    for b in range(NBUF):
        j = (OUTER - 1) * NBUF + b
        wait_write(j, b)


def _select_transpose_body(pairs_ref, out_ref):
    pairs = pairs_ref[0]
    pt = jnp.transpose(pairs, (1, 0))
    out_ref[0] = pt[:D, :]


_select_transpose = pl.pallas_call(
    _select_transpose_body,
    grid=(SEQ,),
    in_specs=[
        pl.BlockSpec((1, B_ROWS, PAIR), lambda t: (t, 0, 0)),
    ],
    out_specs=pl.BlockSpec((1, D, B_ROWS), lambda t: (t, 0, 0)),
    out_shape=jax.ShapeDtypeStruct((SEQ, D, B_ROWS), jnp.float32),
)


def kernel(ids, emb_weight):
    ids_t = jnp.transpose(ids).astype(jnp.int32)
    tnat = jnp.transpose(emb_weight)
    table128 = _prep_table(tnat)
    table2 = table128.reshape(2 * VOCAB, D)
    rows = _gather_rows((ids_t * 2).reshape(-1), table2)
    planes = rows.reshape(SEQ, B_ROWS, PAIR)
    out_t = _select_transpose(planes)
    return out_t.transpose(2, 0, 1)


# speedup vs baseline: 16.5068x; 1.0002x over previous
import functools

import jax
import jax.numpy as jnp
from jax import lax
from jax.experimental import pallas as pl
from jax.experimental.pallas import tpu as pltpu
from jax.experimental.pallas import tpu_sc as plsc

B_ROWS = 4096
SEQ = 200
D = 64
PAIR = 2 * D
VOCAB = 1000000
B_TOTAL = B_ROWS * SEQ

NUM_CORES = 2
NUM_SUBCORES = 16
NW = NUM_CORES * NUM_SUBCORES
PER_W = B_TOTAL // NW
CHUNK = 256
N_CHUNKS = PER_W // CHUNK
NBUF = 4
OUTER = N_CHUNKS // NBUF

_mesh = plsc.VectorSubcoreMesh(core_axis_name="c", subcore_axis_name="s")


_PREP_BK = 16384
_PREP_GRID = -(-VOCAB // _PREP_BK)


def _prep_body(tnat_ref, out_ref):
    block = tnat_ref[...]
    out_ref[:, :D] = jnp.transpose(block, (1, 0))
    out_ref[:, D:] = jnp.zeros((_PREP_BK, D), jnp.float32)


_prep_table = pl.pallas_call(
    _prep_body,
    grid=(_PREP_GRID,),
    in_specs=[pl.BlockSpec((D, _PREP_BK), lambda i: (0, i))],
    out_specs=pl.BlockSpec((_PREP_BK, PAIR), lambda i: (i, 0)),
    out_shape=jax.ShapeDtypeStruct((VOCAB, PAIR), jnp.float32),
)


@functools.partial(
    pl.kernel,
    mesh=_mesh,
    out_type=jax.ShapeDtypeStruct((B_TOTAL, 2, D), jnp.float32),
    scratch_types=(
        [pltpu.VMEM((PER_W,), jnp.int32)]
        + [pltpu.VMEM((CHUNK, D), jnp.float32) for _ in range(NBUF)]
        + [pltpu.SemaphoreType.DMA for _ in range(2 * NBUF)]
    ),
    compiler_params=pltpu.CompilerParams(use_tc_tiling_on_sc=False),
)
def _gather_rows(ids_hbm, table_hbm, out_hbm, idx_v, *bufs_and_sems):
    rows = bufs_and_sems[:NBUF]
    sg = bufs_and_sems[NBUF : 2 * NBUF]
    sw = bufs_and_sems[2 * NBUF : 3 * NBUF]

    wid = lax.axis_index("s") * NUM_CORES + lax.axis_index("c")
    base = wid * PER_W

    pltpu.sync_copy(ids_hbm.at[pl.ds(base, PER_W)], idx_v)

    def start_gather(j, b):
        pltpu.async_copy(
            table_hbm.at[idx_v.at[pl.ds(j * CHUNK, CHUNK)]], rows[b], sg[b]
        )

    def wait_gather(j, b):
        pltpu.make_async_copy(
            table_hbm.at[idx_v.at[pl.ds(j * CHUNK, CHUNK)]], rows[b], sg[b]
        ).wait()

    def start_write(j, b):
        pltpu.async_copy(
            rows[b], out_hbm.at[pl.ds(base + j * CHUNK, CHUNK), 0], sw[b]
        )

    def wait_write(j, b):
        pltpu.make_async_copy(
            rows[b], out_hbm.at[pl.ds(base + j * CHUNK, CHUNK), 0], sw[b]
        ).wait()

    for b in range(NBUF):
        start_gather(b, b)

    def outer(o, carry):
        for b in range(NBUF):
            j = o * NBUF + b
            wait_gather(j, b)
            start_write(j, b)
            wait_write(j, b)
            start_gather(j + NBUF, b)
        return carry

    lax.fori_loop(0, OUTER - 1, outer, 0)

    for b in range(NBUF):
        j = (OUTER - 1) * NBUF + b
        wait_gather(j, b)
        start_write(j, b)
    for b in range(NBUF):
        j = (OUTER - 1) * NBUF + b
        wait_write(j, b)


def _select_transpose_body(pairs_ref, out_ref):
    sel = pairs_ref[0][:, :D]
    out_ref[0] = jnp.transpose(sel, (1, 0))


_select_transpose = pl.pallas_call(
    _select_transpose_body,
    grid=(SEQ,),
    in_specs=[
        pl.BlockSpec((1, B_ROWS, PAIR), lambda t: (t, 0, 0)),
    ],
    out_specs=pl.BlockSpec((1, D, B_ROWS), lambda t: (t, 0, 0)),
    out_shape=jax.ShapeDtypeStruct((SEQ, D, B_ROWS), jnp.float32),
)


def kernel(ids, emb_weight):
    ids_t = jnp.transpose(ids).astype(jnp.int32)
    tnat = jnp.transpose(emb_weight)
    table128 = _prep_table(tnat)
    table2 = table128.reshape(2 * VOCAB, D)
    rows = _gather_rows((ids_t * 2).reshape(-1), table2)
    planes = rows.reshape(SEQ, B_ROWS, PAIR)
    out_t = _select_transpose(planes)
    return out_t.transpose(2, 0, 1)
